# Initial kernel scaffold; baseline (speedup 1.0000x reference)
import jax
import jax.numpy as jnp
from jax.experimental import pallas as pl


def kernel(x, edge_index, edge_weight, W_fc, W_attn):
    raise NotImplementedError("write your pallas kernel here")


# speedup vs baseline: 13.0737x; 13.0737x over previous
import functools

import jax
import jax.numpy as jnp
from jax import lax
from jax.experimental import pallas as pl
from jax.experimental.pallas import tpu as pltpu
from jax.experimental.pallas import tpu_sc as plsc

NC = 2
NS = 16
NW = NC * NS
LANES = 16
CHUNK = 128


def _pre_body(x_ref, w_ref, al_ref, ar_ref, z_ref, sl_ref, sr_ref, cv_ref):
    x = x_ref[...]
    z = lax.dot_general(x, w_ref[...], (((1,), (1,)), ((), ())),
                        preferred_element_type=jnp.float32)
    z_ref[...] = z
    sl = jnp.sum(z * al_ref[...][None, :], axis=1)
    sr = jnp.sum(z * ar_ref[...][None, :], axis=1)
    sl_ref[...] = sl
    sr_ref[...] = sr
    c_sh = jnp.maximum(jnp.max(sl) + jnp.max(sr), 0.0)
    cv_ref[...] = jnp.full((LANES,), c_sh, jnp.float32)


def _post_body(n, hp_ref, dp_ref, o_ref):
    den = dp_ref[0, :n] + dp_ref[1, :n]
    den = jnp.where(den == 0.0, 1.0, den)
    h = hp_ref[0, :n, :] + hp_ref[1, :n, :]
    o_ref[...] = h / den[:, None]


def _make_sc_kernel(n, d, n_pad, ch):
    rows_per_tile = n_pad // NS
    zcopies = rows_per_tile // CHUNK

    mesh = plsc.VectorSubcoreMesh(core_axis_name="c", subcore_axis_name="s")

    @functools.partial(
        pl.kernel,
        out_type=[
            jax.ShapeDtypeStruct((NC, n_pad, d), jnp.float32),
            jax.ShapeDtypeStruct((NC, n_pad), jnp.float32),
        ],
        mesh=mesh,
        scratch_types=[
            pltpu.VMEM((CHUNK,), jnp.float32),
            pltpu.VMEM((CHUNK,), jnp.float32),
            pltpu.VMEM((ch, CHUNK), jnp.int32),
            pltpu.VMEM((ch, CHUNK), jnp.int32),
            pltpu.VMEM((CHUNK,), jnp.float32),
            pltpu.VMEM((CHUNK,), jnp.float32),
            pltpu.VMEM((CHUNK, d), jnp.float32),
            pltpu.VMEM((rows_per_tile,), jnp.float32),
            pltpu.VMEM((LANES,), jnp.float32),
            pltpu.VMEM_SHARED((n_pad, d), jnp.float32),
            pltpu.VMEM_SHARED((n_pad,), jnp.float32),
            pltpu.SemaphoreType.DMA,
        ],
    )
    def sc_kernel(z_hbm, sl_hbm, sr_hbm, src_hbm, dst_hbm, w_hbm, cv_hbm,
                  h_out, den_out,
                  slg_v, srg_v, src_v, dst_v, w_c, ex_c, rows_v, zden_v, cv_v,
                  h_sh, den_sh, sem):
        c = lax.axis_index("c")
        s = lax.axis_index("s")
        w_id = c * NS + s
        base = s * rows_per_tile

        pltpu.sync_copy(src_hbm.at[w_id], src_v)
        pltpu.sync_copy(dst_hbm.at[w_id], dst_v)
        pltpu.sync_copy(cv_hbm, cv_v)

        def zrow(r, _):
            for f in range(d // LANES):
                rows_v[r, pl.ds(f * LANES, LANES)] = jnp.zeros(
                    (LANES,), jnp.float32)
            return 0
        lax.fori_loop(0, CHUNK, zrow, 0)
        def zden(i, _):
            zden_v[pl.ds(i * LANES, LANES)] = jnp.zeros((LANES,), jnp.float32)
            return 0
        lax.fori_loop(0, rows_per_tile // LANES, zden, 0)
        for b in range(zcopies):
            pltpu.sync_copy(rows_v,
                            h_sh.at[pl.ds(base + b * CHUNK, CHUNK)])
        pltpu.sync_copy(zden_v, den_sh.at[pl.ds(base, rows_per_tile)])

        c_sh = cv_v[...]

        plsc.subcore_barrier()

        def cbody(j, _):
            pltpu.async_copy(sl_hbm.at[src_v.at[j]], slg_v, sem).wait()
            pltpu.async_copy(sr_hbm.at[dst_v.at[j]], srg_v, sem).wait()
            pltpu.sync_copy(w_hbm.at[w_id, j], w_c)
            for k in range(CHUNK // LANES):
                wk = w_c[pl.ds(k * LANES, LANES)]
                raw = (slg_v[pl.ds(k * LANES, LANES)]
                       + srg_v[pl.ds(k * LANES, LANES)])
                e = wk * jnp.maximum(raw, 0.01 * raw)
                ex = jnp.where(wk >= 0.0, jnp.exp(e - c_sh), 0.0)
                ex_c[pl.ds(k * LANES, LANES)] = ex
            pltpu.sync_copy(ex_c, den_sh.at[dst_v.at[j]], add=True)
            pltpu.async_copy(z_hbm.at[src_v.at[j]], rows_v, sem).wait()
            def rblk(k, _):
                exk = ex_c[pl.ds(k * LANES, LANES)]
                for r in range(LANES):
                    a = exk[r]
                    row = k * LANES + r
                    for f in range(d // LANES):
                        v = rows_v[row, pl.ds(f * LANES, LANES)]
                        rows_v[row, pl.ds(f * LANES, LANES)] = v * a
                return 0
            lax.fori_loop(0, CHUNK // LANES, rblk, 0)
            pltpu.sync_copy(rows_v, h_sh.at[dst_v.at[j]], add=True)
            return 0
        lax.fori_loop(0, ch, cbody, 0)

        plsc.subcore_barrier()

        pltpu.sync_copy(h_sh.at[pl.ds(base, rows_per_tile)],
                        h_out.at[c, pl.ds(base, rows_per_tile)])
        pltpu.sync_copy(den_sh.at[pl.ds(base, rows_per_tile)],
                        den_out.at[c, pl.ds(base, rows_per_tile)])

    return sc_kernel


def kernel(x, edge_index, edge_weight, W_fc, W_attn):
    n, d_in = x.shape
    d = W_fc.shape[0]
    e_cnt = edge_index.shape[1]
    assert n % LANES == 0 and d % LANES == 0

    a_l = W_attn[0, :d]
    a_r = W_attn[0, d:]

    z, sl, sr, cvec = pl.pallas_call(
        _pre_body,
        out_shape=[
            jax.ShapeDtypeStruct((n, d), jnp.float32),
            jax.ShapeDtypeStruct((n,), jnp.float32),
            jax.ShapeDtypeStruct((n,), jnp.float32),
            jax.ShapeDtypeStruct((LANES,), jnp.float32),
        ],
    )(x, W_fc, a_l, a_r)

    ch = -(-e_cnt // (NW * CHUNK))
    e_pad = NW * ch * CHUNK
    src = jnp.pad(edge_index[0], (0, e_pad - e_cnt)).reshape(NW, ch, CHUNK)
    dst = jnp.pad(edge_index[1], (0, e_pad - e_cnt)).reshape(NW, ch, CHUNK)
    wgt = jnp.pad(edge_weight, (0, e_pad - e_cnt),
                  constant_values=-1.0).reshape(NW, ch, CHUNK)

    n_pad = -(-n // (NS * CHUNK)) * NS * CHUNK
    hp, dp = _make_sc_kernel(n, d, n_pad, ch)(z, sl, sr, src, dst, wgt, cvec)

    out = pl.pallas_call(
        functools.partial(_post_body, n),
        out_shape=jax.ShapeDtypeStruct((n, d), jnp.float32),
    )(hp, dp)
    return out
